# baseline (speedup 1.0000x reference)
import functools

import jax
import jax.numpy as jnp
from jax import lax
from jax.experimental import pallas as pl
from jax.experimental.pallas import tpu as pltpu
from jax.experimental.pallas import tpu_sc as plsc

_LANES = 16
_CH = 128
_NCORES = 2
_NSUB = 16


_GATHER_DNUMS = lax.GatherDimensionNumbers(
    offset_dims=(), collapsed_slice_dims=(0,), start_index_map=(0,))


def _lane_gather(vec, idx):
    return lax.gather(vec, idx, _GATHER_DNUMS, slice_sizes=(1,),
                      mode=lax.GatherScatterMode.PROMISE_IN_BOUNDS)


def _bcast_lane(vec, j):
    return _lane_gather(vec, jnp.full((_LANES, 1), j, dtype=jnp.int32))


def _make_edge_kernel(npad, e, with_cnt):
    w = 32 if with_cnt else _LANES
    tw = 32 if with_cnt else _LANES
    nstr = 5
    chunk = nstr * _CH
    ntot = e // chunk
    nw = _NCORES * _NSUB
    nt = -(-ntot // nw)
    nt += nt % 2
    rps = npad // _NSUB
    assert chunk >= rps
    mesh = plsc.VectorSubcoreMesh(core_axis_name="c", subcore_axis_name="s")

    slot_scratch = [
        pltpu.VMEM((chunk,), jnp.int32),
        *[pltpu.VMEM((_CH,), jnp.int32) for _ in range(nstr)],
        pltpu.VMEM((chunk,), jnp.float32),
        pltpu.VMEM((chunk, tw), jnp.float32),
        pltpu.VMEM((chunk, w), jnp.float32),
        pltpu.SemaphoreType.DMA,
        pltpu.SemaphoreType.DMA,
        pltpu.SemaphoreType.DMA,
        *[pltpu.SemaphoreType.DMA for _ in range(nstr)],
        pltpu.SemaphoreType.DMA,
    ]
    nslot = len(slot_scratch)

    @functools.partial(
        pl.kernel,
        mesh=mesh,
        compiler_params=pltpu.CompilerParams(use_tc_tiling_on_sc=False),
        out_type=jax.ShapeDtypeStruct((2 * npad, w), jnp.float32),
        scratch_types=[
            *slot_scratch, *slot_scratch,
            pltpu.VMEM_SHARED((npad, w), jnp.float32),
        ],
    )
    def edge_kernel(tab_hbm, src_hbm, dst_hbm, u_hbm, out_hbm, *sc):
        slots = [sc[:nslot], sc[nslot:2 * nslot]]
        acc = sc[2 * nslot]
        cid = lax.axis_index("c")
        sid = lax.axis_index("s")
        wid = sid * _NCORES + cid
        r0 = sid * rps

        def parts(slot):
            return dict(
                srcv=slot[0], dst=slot[1:1 + nstr], uv=slot[1 + nstr],
                rows=slot[2 + nstr], msg=slot[3 + nstr], gsem=slot[4 + nstr],
                ssrc=slot[5 + nstr], su=slot[6 + nstr],
                sdst=slot[7 + nstr:7 + 2 * nstr], ssc=slot[7 + 2 * nstr])

        def chunk_base(t):
            c = wid + nw * t
            return jnp.where(c < ntot, c, wid) * chunk

        def idx_copies(slot, t):
            p = parts(slot)
            base = chunk_base(t)
            cps = [
                (src_hbm.at[pl.ds(base, chunk)], p["srcv"], p["ssrc"]),
                (u_hbm.at[pl.ds(base, chunk)], p["uv"], p["su"]),
            ]
            for k in range(nstr):
                cps.append((dst_hbm.at[pl.ds(base + k * _CH, _CH)],
                            p["dst"][k], p["sdst"][k]))
            return cps

        def idx_issue(slot, t):
            for a, b, s in idx_copies(slot, t):
                pltpu.async_copy(a, b, s)

        def idx_wait(slot, t):
            for a, b, s in idx_copies(slot, t):
                pltpu.make_async_copy(a, b, s).wait()

        def gather_copies(slot):
            p = parts(slot)
            return [
                (tab_hbm.at[p["srcv"].at[pl.ds(k * _CH, _CH)]],
                 p["rows"].at[pl.ds(k * _CH, _CH)], p["gsem"])
                for k in range(nstr)
            ]

        def gather_issue(slot):
            for a, b, s in gather_copies(slot):
                pltpu.async_copy(a, b, s)

        def gather_wait(slot):
            for a, b, s in gather_copies(slot):
                pltpu.make_async_copy(a, b, s).wait()

        umask = lax.iota(jnp.int32, _LANES) >= 8

        def combine(slot):
            p = parts(slot)
            uv, rows, msg = p["uv"], p["rows"], p["msg"]

            @pl.loop(0, chunk, step=_LANES)
            def _(g):
                uvec = uv[pl.ds(g, _LANES)]
                for j in range(_LANES):
                    ub = _bcast_lane(uvec, j)
                    if tw == 32:
                        lo = rows[g + j, pl.ds(0, _LANES)]
                        hi = rows[g + j, pl.ds(_LANES, _LANES)]
                        msg[g + j, pl.ds(0, _LANES)] = lo + ub * hi
                    else:
                        row = rows[g + j, pl.ds(0, _LANES)]
                        msg[g + j, pl.ds(0, _LANES)] = row * jnp.where(
                            umask, ub, jnp.float32(1.0))

        def scatter(slot):
            p = parts(slot)
            cps = [(p["msg"].at[pl.ds(k * _CH, _CH)], acc.at[p["dst"][k]])
                   for k in range(nstr)]
            for a, b in cps:
                pltpu.async_copy(a, b, p["ssc"], add=True)
            for a, b in cps:
                pltpu.make_async_copy(a, b, p["ssc"]).wait()

        zero = jnp.zeros((_LANES,), jnp.float32)
        zmsg = parts(slots[0])["msg"]

        @pl.loop(0, chunk)
        def _(r):
            for h in range(w // _LANES):
                zmsg[r, pl.ds(h * _LANES, _LANES)] = zero

        pltpu.sync_copy(zmsg.at[pl.ds(0, rps)], acc.at[pl.ds(r0, rps)])

        if with_cnt:
            one = jnp.ones((_LANES,), jnp.float32)
            for slot in slots:
                msg = parts(slot)["msg"]

                @pl.loop(0, chunk)
                def _(r):
                    msg[r, pl.ds(_LANES, _LANES)] = one

        plsc.subcore_barrier()

        idx_issue(slots[0], 0)
        idx_wait(slots[0], 0)
        gather_issue(slots[0])
        idx_issue(slots[1], 1)

        @pl.loop(0, nt, step=2)
        def _(tt):
            for b in range(2):
                t = tt + b
                cur, nxt = slots[b], slots[1 - b]
                idx_wait(nxt, t + 1)
                gather_issue(nxt)
                gather_wait(cur)
                combine(cur)

                @pl.when(wid + nw * t < ntot)
                def _():
                    scatter(cur)

                idx_issue(cur, t + 2)

        gather_wait(slots[nt % 2])
        idx_wait(slots[(nt + 1) % 2], nt + 1)

        plsc.subcore_barrier()
        pltpu.sync_copy(acc.at[pl.ds(r0, rps)],
                        out_hbm.at[pl.ds(cid * npad + r0, rps)])

    return edge_kernel


def _dense_pre(x, wc, rc, b):
    n, f = x.shape
    blk = 1000

    def body(x_ref, wc_ref, rc_ref, b_ref, p_ref, r_ref):
        xb = x_ref[...]
        p_ref[...] = jnp.dot(xb, wc_ref[...],
                             preferred_element_type=jnp.float32,
                             precision=lax.Precision.HIGHEST)
        r_ref[...] = jnp.dot(xb, rc_ref[...],
                             preferred_element_type=jnp.float32,
                             precision=lax.Precision.HIGHEST) + b_ref[...]

    return pl.pallas_call(
        body,
        grid=(n // blk,),
        in_specs=[
            pl.BlockSpec((blk, f), lambda i: (i, 0)),
            pl.BlockSpec((f, 32), lambda i: (0, 0)),
            pl.BlockSpec((f, 16), lambda i: (0, 0)),
            pl.BlockSpec((1, 16), lambda i: (0, 0)),
        ],
        out_specs=[
            pl.BlockSpec((blk, 32), lambda i: (i, 0)),
            pl.BlockSpec((blk, 16), lambda i: (i, 0)),
        ],
        out_shape=[
            jax.ShapeDtypeStruct((n, 32), jnp.float32),
            jax.ShapeDtypeStruct((n, 16), jnp.float32),
        ],
    )(x, wc, rc, b)


def _dense_mid(acc_a, acc_b, r1, wc2, rc2, b2):
    n = r1.shape[0]
    blk = 1000

    def body(a_ref, b_ref, r1_ref, wc2_ref, rc2_ref, b2_ref,
             p2_ref, r2_ref, inv_ref):
        a = a_ref[...] + b_ref[...]
        agg = a[:, :16]
        cnt = a[:, 16:17]
        inv = 1.0 / jnp.maximum(cnt, 1.0)
        v = agg * inv + r1_ref[...]
        h = jnp.where(v > 0, v, jnp.exp(v) - 1.0)
        p2_ref[...] = jnp.dot(h, wc2_ref[...],
                              preferred_element_type=jnp.float32,
                              precision=lax.Precision.HIGHEST)
        r2_ref[...] = jnp.dot(h, rc2_ref[...],
                              preferred_element_type=jnp.float32,
                              precision=lax.Precision.HIGHEST) + b2_ref[...]
        inv_ref[...] = jnp.broadcast_to(inv, (inv.shape[0], 8))

    return pl.pallas_call(
        body,
        grid=(n // blk,),
        in_specs=[
            pl.BlockSpec((blk, 32), lambda i: (i, 0)),
            pl.BlockSpec((blk, 32), lambda i: (i, 0)),
            pl.BlockSpec((blk, 16), lambda i: (i, 0)),
            pl.BlockSpec((16, 16), lambda i: (0, 0)),
            pl.BlockSpec((16, 8), lambda i: (0, 0)),
            pl.BlockSpec((1, 8), lambda i: (0, 0)),
        ],
        out_specs=[
            pl.BlockSpec((blk, 16), lambda i: (i, 0)),
            pl.BlockSpec((blk, 8), lambda i: (i, 0)),
            pl.BlockSpec((blk, 8), lambda i: (i, 0)),
        ],
        out_shape=[
            jax.ShapeDtypeStruct((n, 16), jnp.float32),
            jax.ShapeDtypeStruct((n, 8), jnp.float32),
            jax.ShapeDtypeStruct((n, 8), jnp.float32),
        ],
    )(acc_a, acc_b, r1, wc2, rc2, b2)


def _dense_final(acc_a, acc_b, inv8, r2, c):
    n = r2.shape[0]
    blk = 1000

    def body(a_ref, b_ref, inv_ref, r2_ref, o_ref):
        a = a_ref[...] + b_ref[...]
        agg = a[:, :8] + a[:, 8:16]
        s = agg[:, :c] * inv_ref[:, :c] + r2_ref[:, :c]
        m = jnp.max(s, axis=1, keepdims=True)
        z = s - m
        o_ref[...] = z - jnp.log(jnp.sum(jnp.exp(z), axis=1, keepdims=True))

    return pl.pallas_call(
        body,
        grid=(n // blk,),
        in_specs=[
            pl.BlockSpec((blk, 16), lambda i: (i, 0)),
            pl.BlockSpec((blk, 16), lambda i: (i, 0)),
            pl.BlockSpec((blk, 8), lambda i: (i, 0)),
            pl.BlockSpec((blk, 8), lambda i: (i, 0)),
        ],
        out_specs=pl.BlockSpec((blk, c), lambda i: (i, 0)),
        out_shape=jax.ShapeDtypeStruct((n, c), jnp.float32),
    )(acc_a, acc_b, inv8, r2)


def kernel(x, edge_index, edge_attr, W1, root1, b1, W2, root2, b2):
    n, f = x.shape
    e = edge_index.shape[1]
    hid = W1.shape[2]
    c = W2.shape[2]
    assert e % (5 * _CH) == 0 and n % _NSUB == 0

    src = edge_index[0].astype(jnp.int32)
    dst = edge_index[1].astype(jnp.int32)
    u = edge_attr[:, 0].astype(jnp.float32)

    npad = ((n + 8 * _NSUB - 1) // (8 * _NSUB)) * (8 * _NSUB)

    wc1 = jnp.concatenate([W1[0], W1[1] - W1[0]], axis=1)
    p1, r1 = _dense_pre(x, wc1, root1, b1.reshape(1, hid))

    acc1 = _make_edge_kernel(npad, e, with_cnt=True)(p1, src, dst, u)

    pad = 8 - c
    wc2 = jnp.concatenate(
        [jnp.pad(W2[0], ((0, 0), (0, pad))),
         jnp.pad(W2[1] - W2[0], ((0, 0), (0, pad)))], axis=1)
    rc2 = jnp.pad(root2, ((0, 0), (0, 8 - c)))
    b2p = jnp.pad(b2, (0, 8 - c)).reshape(1, 8)
    p2, r2, inv8 = _dense_mid(acc1[:n], acc1[npad:npad + n], r1,
                              wc2, rc2, b2p)

    acc2 = _make_edge_kernel(npad, e, with_cnt=False)(p2, src, dst, u)

    return _dense_final(acc2[:n], acc2[npad:npad + n], inv8, r2, c)


# speedup vs baseline: 1.0092x; 1.0092x over previous
import functools

import jax
import jax.numpy as jnp
from jax import lax
from jax.experimental import pallas as pl
from jax.experimental.pallas import tpu as pltpu
from jax.experimental.pallas import tpu_sc as plsc

_LANES = 16
_CH = 128
_NCORES = 2
_NSUB = 16


_GATHER_DNUMS = lax.GatherDimensionNumbers(
    offset_dims=(), collapsed_slice_dims=(0,), start_index_map=(0,))


def _lane_gather(vec, idx):
    return lax.gather(vec, idx, _GATHER_DNUMS, slice_sizes=(1,),
                      mode=lax.GatherScatterMode.PROMISE_IN_BOUNDS)


def _bcast_lane(vec, j):
    return _lane_gather(vec, jnp.full((_LANES, 1), j, dtype=jnp.int32))


def _make_edge_kernel(npad, e, with_cnt):
    w = 32 if with_cnt else _LANES
    tw = 32 if with_cnt else _LANES
    nstr = 5 if with_cnt else 10
    chunk = nstr * _CH
    ntot = e // chunk
    nw = _NCORES * _NSUB
    nt = -(-ntot // nw)
    nt += nt % 2
    rps = npad // _NSUB
    mesh = plsc.VectorSubcoreMesh(core_axis_name="c", subcore_axis_name="s")

    slot_scratch = [
        pltpu.VMEM((chunk,), jnp.int32),
        *[pltpu.VMEM((_CH,), jnp.int32) for _ in range(nstr)],
        pltpu.VMEM((chunk,), jnp.float32),
        pltpu.VMEM((chunk, tw), jnp.float32),
        pltpu.VMEM((chunk, w), jnp.float32),
        pltpu.SemaphoreType.DMA,
        pltpu.SemaphoreType.DMA,
        pltpu.SemaphoreType.DMA,
        *[pltpu.SemaphoreType.DMA for _ in range(nstr)],
        pltpu.SemaphoreType.DMA,
    ]
    nslot = len(slot_scratch)

    @functools.partial(
        pl.kernel,
        mesh=mesh,
        compiler_params=pltpu.CompilerParams(use_tc_tiling_on_sc=False),
        out_type=jax.ShapeDtypeStruct((2 * npad, w), jnp.float32),
        scratch_types=[
            *slot_scratch, *slot_scratch,
            pltpu.VMEM_SHARED((npad, w), jnp.float32),
        ],
    )
    def edge_kernel(tab_hbm, src_hbm, dst_hbm, u_hbm, z_hbm, out_hbm, *sc):
        slots = [sc[:nslot], sc[nslot:2 * nslot]]
        acc = sc[2 * nslot]
        cid = lax.axis_index("c")
        sid = lax.axis_index("s")
        wid = sid * _NCORES + cid
        r0 = sid * rps

        def parts(slot):
            return dict(
                srcv=slot[0], dst=slot[1:1 + nstr], uv=slot[1 + nstr],
                rows=slot[2 + nstr], msg=slot[3 + nstr], gsem=slot[4 + nstr],
                ssrc=slot[5 + nstr], su=slot[6 + nstr],
                sdst=slot[7 + nstr:7 + 2 * nstr], ssc=slot[7 + 2 * nstr])

        def chunk_base(t):
            c = wid + nw * t
            return jnp.where(c < ntot, c, wid) * chunk

        def idx_copies(slot, t):
            p = parts(slot)
            base = chunk_base(t)
            cps = [
                (src_hbm.at[pl.ds(base, chunk)], p["srcv"], p["ssrc"]),
                (u_hbm.at[pl.ds(base, chunk)], p["uv"], p["su"]),
            ]
            for k in range(nstr):
                cps.append((dst_hbm.at[pl.ds(base + k * _CH, _CH)],
                            p["dst"][k], p["sdst"][k]))
            return cps

        def idx_issue(slot, t):
            for a, b, s in idx_copies(slot, t):
                pltpu.async_copy(a, b, s)

        def idx_wait(slot, t):
            for a, b, s in idx_copies(slot, t):
                pltpu.make_async_copy(a, b, s).wait()

        def gather_copies(slot):
            p = parts(slot)
            return [
                (tab_hbm.at[p["srcv"].at[pl.ds(k * _CH, _CH)]],
                 p["rows"].at[pl.ds(k * _CH, _CH)], p["gsem"])
                for k in range(nstr)
            ]

        def gather_issue(slot):
            for a, b, s in gather_copies(slot):
                pltpu.async_copy(a, b, s)

        def gather_wait(slot):
            for a, b, s in gather_copies(slot):
                pltpu.make_async_copy(a, b, s).wait()

        umask = lax.iota(jnp.int32, _LANES) >= 8

        def combine(slot):
            p = parts(slot)
            uv, rows, msg = p["uv"], p["rows"], p["msg"]

            @pl.loop(0, chunk, step=_LANES)
            def _(g):
                uvec = uv[pl.ds(g, _LANES)]
                for j in range(_LANES):
                    ub = _bcast_lane(uvec, j)
                    if tw == 32:
                        lo = rows[g + j, pl.ds(0, _LANES)]
                        hi = rows[g + j, pl.ds(_LANES, _LANES)]
                        msg[g + j, pl.ds(0, _LANES)] = lo + ub * hi
                    else:
                        row = rows[g + j, pl.ds(0, _LANES)]
                        msg[g + j, pl.ds(0, _LANES)] = row * jnp.where(
                            umask, ub, jnp.float32(1.0))

        def scatter(slot):
            p = parts(slot)
            cps = [(p["msg"].at[pl.ds(k * _CH, _CH)], acc.at[p["dst"][k]])
                   for k in range(nstr)]
            for a, b in cps:
                pltpu.async_copy(a, b, p["ssc"], add=True)
            for a, b in cps:
                pltpu.make_async_copy(a, b, p["ssc"]).wait()

        pltpu.sync_copy(z_hbm.at[pl.ds(r0, rps)], acc.at[pl.ds(r0, rps)])

        if with_cnt:
            one = jnp.ones((_LANES,), jnp.float32)
            for slot in slots:
                msg = parts(slot)["msg"]

                @pl.loop(0, chunk)
                def _(r):
                    msg[r, pl.ds(_LANES, _LANES)] = one

        plsc.subcore_barrier()

        idx_issue(slots[0], 0)
        idx_wait(slots[0], 0)
        gather_issue(slots[0])
        idx_issue(slots[1], 1)

        @pl.loop(0, nt, step=2)
        def _(tt):
            for b in range(2):
                t = tt + b
                cur, nxt = slots[b], slots[1 - b]
                idx_wait(nxt, t + 1)
                gather_issue(nxt)
                gather_wait(cur)
                combine(cur)

                @pl.when(wid + nw * t < ntot)
                def _():
                    scatter(cur)

                idx_issue(cur, t + 2)

        gather_wait(slots[nt % 2])
        idx_wait(slots[(nt + 1) % 2], nt + 1)

        plsc.subcore_barrier()
        pltpu.sync_copy(acc.at[pl.ds(r0, rps)],
                        out_hbm.at[pl.ds(cid * npad + r0, rps)])

    return edge_kernel


def _dense_pre(x, wc, rc, b):
    n, f = x.shape
    blk = 1000

    def body(x_ref, wc_ref, rc_ref, b_ref, p_ref, r_ref):
        xb = x_ref[...]
        p_ref[...] = jnp.dot(xb, wc_ref[...],
                             preferred_element_type=jnp.float32,
                             precision=lax.Precision.HIGHEST)
        r_ref[...] = jnp.dot(xb, rc_ref[...],
                             preferred_element_type=jnp.float32,
                             precision=lax.Precision.HIGHEST) + b_ref[...]

    return pl.pallas_call(
        body,
        grid=(n // blk,),
        in_specs=[
            pl.BlockSpec((blk, f), lambda i: (i, 0)),
            pl.BlockSpec((f, 32), lambda i: (0, 0)),
            pl.BlockSpec((f, 16), lambda i: (0, 0)),
            pl.BlockSpec((1, 16), lambda i: (0, 0)),
        ],
        out_specs=[
            pl.BlockSpec((blk, 32), lambda i: (i, 0)),
            pl.BlockSpec((blk, 16), lambda i: (i, 0)),
        ],
        out_shape=[
            jax.ShapeDtypeStruct((n, 32), jnp.float32),
            jax.ShapeDtypeStruct((n, 16), jnp.float32),
        ],
    )(x, wc, rc, b)


def _dense_mid(acc_a, acc_b, r1, wc2, rc2, b2):
    n = r1.shape[0]
    blk = 1000

    def body(a_ref, b_ref, r1_ref, wc2_ref, rc2_ref, b2_ref,
             p2_ref, r2_ref, inv_ref):
        a = a_ref[...] + b_ref[...]
        agg = a[:, :16]
        cnt = a[:, 16:17]
        inv = 1.0 / jnp.maximum(cnt, 1.0)
        v = agg * inv + r1_ref[...]
        h = jnp.where(v > 0, v, jnp.exp(v) - 1.0)
        p2_ref[...] = jnp.dot(h, wc2_ref[...],
                              preferred_element_type=jnp.float32,
                              precision=lax.Precision.HIGHEST)
        r2_ref[...] = jnp.dot(h, rc2_ref[...],
                              preferred_element_type=jnp.float32,
                              precision=lax.Precision.HIGHEST) + b2_ref[...]
        inv_ref[...] = jnp.broadcast_to(inv, (inv.shape[0], 8))

    return pl.pallas_call(
        body,
        grid=(n // blk,),
        in_specs=[
            pl.BlockSpec((blk, 32), lambda i: (i, 0)),
            pl.BlockSpec((blk, 32), lambda i: (i, 0)),
            pl.BlockSpec((blk, 16), lambda i: (i, 0)),
            pl.BlockSpec((16, 16), lambda i: (0, 0)),
            pl.BlockSpec((16, 8), lambda i: (0, 0)),
            pl.BlockSpec((1, 8), lambda i: (0, 0)),
        ],
        out_specs=[
            pl.BlockSpec((blk, 16), lambda i: (i, 0)),
            pl.BlockSpec((blk, 8), lambda i: (i, 0)),
            pl.BlockSpec((blk, 8), lambda i: (i, 0)),
        ],
        out_shape=[
            jax.ShapeDtypeStruct((n, 16), jnp.float32),
            jax.ShapeDtypeStruct((n, 8), jnp.float32),
            jax.ShapeDtypeStruct((n, 8), jnp.float32),
        ],
    )(acc_a, acc_b, r1, wc2, rc2, b2)


def _dense_final(acc_a, acc_b, inv8, r2, c):
    n = r2.shape[0]
    blk = 1000

    def body(a_ref, b_ref, inv_ref, r2_ref, o_ref):
        a = a_ref[...] + b_ref[...]
        agg = a[:, :8] + a[:, 8:16]
        s = agg[:, :c] * inv_ref[:, :c] + r2_ref[:, :c]
        m = jnp.max(s, axis=1, keepdims=True)
        z = s - m
        o_ref[...] = z - jnp.log(jnp.sum(jnp.exp(z), axis=1, keepdims=True))

    return pl.pallas_call(
        body,
        grid=(n // blk,),
        in_specs=[
            pl.BlockSpec((blk, 16), lambda i: (i, 0)),
            pl.BlockSpec((blk, 16), lambda i: (i, 0)),
            pl.BlockSpec((blk, 8), lambda i: (i, 0)),
            pl.BlockSpec((blk, 8), lambda i: (i, 0)),
        ],
        out_specs=pl.BlockSpec((blk, c), lambda i: (i, 0)),
        out_shape=jax.ShapeDtypeStruct((n, c), jnp.float32),
    )(acc_a, acc_b, inv8, r2)


def kernel(x, edge_index, edge_attr, W1, root1, b1, W2, root2, b2):
    n, f = x.shape
    e = edge_index.shape[1]
    hid = W1.shape[2]
    c = W2.shape[2]
    assert e % (10 * _CH) == 0 and n % _NSUB == 0

    src = edge_index[0].astype(jnp.int32)
    dst = edge_index[1].astype(jnp.int32)
    u = edge_attr[:, 0].astype(jnp.float32)

    npad = ((n + 8 * _NSUB - 1) // (8 * _NSUB)) * (8 * _NSUB)

    wc1 = jnp.concatenate([W1[0], W1[1] - W1[0]], axis=1)
    p1, r1 = _dense_pre(x, wc1, root1, b1.reshape(1, hid))

    z32 = jnp.zeros((npad, 32), jnp.float32)
    acc1 = _make_edge_kernel(npad, e, with_cnt=True)(p1, src, dst, u, z32)

    pad = 8 - c
    wc2 = jnp.concatenate(
        [jnp.pad(W2[0], ((0, 0), (0, pad))),
         jnp.pad(W2[1] - W2[0], ((0, 0), (0, pad)))], axis=1)
    rc2 = jnp.pad(root2, ((0, 0), (0, 8 - c)))
    b2p = jnp.pad(b2, (0, 8 - c)).reshape(1, 8)
    p2, r2, inv8 = _dense_mid(acc1[:n], acc1[npad:npad + n], r1,
                              wc2, rc2, b2p)

    z16 = jnp.zeros((npad, _LANES), jnp.float32)
    acc2 = _make_edge_kernel(npad, e, with_cnt=False)(p2, src, dst, u, z16)

    return _dense_final(acc2[:n], acc2[npad:npad + n], inv8, r2, c)
